# baseline (speedup 1.0000x reference)
import functools

import jax
import jax.numpy as jnp
from jax.experimental import pallas as pl
from jax.experimental.pallas import tpu as pltpu

N = 10000
D = 128

_BI = 400
_BR = 2000


def _relu(h):
    return jnp.maximum(h, 0.0)


def _elu(h):
    return jnp.where(h > 0, h, jnp.exp(jnp.minimum(h, 0.0)) - 1.0)


def _sigmoid(h):
    return jax.nn.sigmoid(h)


def _y0_kernel(x_ref, w_ref, o_ref):
    o_ref[...] = jnp.dot(
        _relu(x_ref[...]), w_ref[...], preferred_element_type=jnp.float32
    ).astype(jnp.bfloat16)


def _y0(x, w):
    return pl.pallas_call(
        _y0_kernel,
        grid=(N // _BR,),
        in_specs=[
            pl.BlockSpec((_BR, D), lambda i: (i, 0)),
            pl.BlockSpec((D, D), lambda i: (0, 0)),
        ],
        out_specs=pl.BlockSpec((_BR, D), lambda i: (i, 0)),
        out_shape=jax.ShapeDtypeStruct((N, D), jnp.bfloat16),
    )(x, w)


def _layer0_kernel(adj_ref, y_ref, b_ref, w2_ref, o_ref, adjb_ref, y1_ref):
    ab = adj_ref[...].astype(jnp.bfloat16)
    adjb_ref[...] = ab
    h = jnp.dot(ab, y_ref[...], preferred_element_type=jnp.float32)
    out = _relu(h + b_ref[...])
    o_ref[...] = out
    y1_ref[...] = jnp.dot(
        out, w2_ref[...], preferred_element_type=jnp.float32
    ).astype(jnp.bfloat16)


def _layer0(adj, y0, b1_2d, w2):
    return pl.pallas_call(
        _layer0_kernel,
        grid=(N // _BI,),
        in_specs=[
            pl.BlockSpec((_BI, N), lambda i: (i, 0)),
            pl.BlockSpec((N, D), lambda i: (0, 0)),
            pl.BlockSpec((1, D), lambda i: (0, 0)),
            pl.BlockSpec((D, D), lambda i: (0, 0)),
        ],
        out_specs=[
            pl.BlockSpec((_BI, D), lambda i: (i, 0)),
            pl.BlockSpec((_BI, N), lambda i: (i, 0)),
            pl.BlockSpec((_BI, D), lambda i: (i, 0)),
        ],
        out_shape=[
            jax.ShapeDtypeStruct((N, D), jnp.float32),
            jax.ShapeDtypeStruct((N, N), jnp.bfloat16),
            jax.ShapeDtypeStruct((N, D), jnp.bfloat16),
        ],
    )(adj, y0, b1_2d, w2)


def _mega_kernel(adjb_ref, y1_ref, out0_ref, wn_ref, b_ref, o_ref,
                 bufs_ref, sems_ref, ya_ref, yb_ref, o1_ref, o2_ref):
    g = pl.program_id(0)
    i = pl.program_id(1)
    nb = N // _BI
    t = g * nb + i
    rows = pl.ds(i * _BI, _BI)

    def copy_for(tt):
        return pltpu.make_async_copy(
            adjb_ref.at[pl.ds((tt % nb) * _BI, _BI), :],
            bufs_ref.at[tt % 3],
            sems_ref.at[tt % 3],
        )

    @pl.when(t == 0)
    def _():
        copy_for(0).start()
        copy_for(1).start()

    @pl.when(t + 2 < 4 * nb)
    def _():
        copy_for(t + 2).start()

    copy_for(t).wait()
    slot = t % 3

    def conv(y_src, act):
        ks = (N // 2) // 128 * 128
        a = bufs_ref[slot]
        if ks == 0:
            h = jnp.dot(a, y_src[...], preferred_element_type=jnp.float32)
        else:
            h = (jnp.dot(a[:, :ks], y_src[:ks, :],
                         preferred_element_type=jnp.float32)
                 + jnp.dot(a[:, ks:], y_src[ks:, :],
                           preferred_element_type=jnp.float32))
        return act(h + b_ref[0])

    def next_y(merge):
        return jnp.dot(_relu(merge), wn_ref[0],
                       preferred_element_type=jnp.float32).astype(jnp.bfloat16)

    @pl.when(g == 0)
    def _():
        out = conv(y1_ref, _elu)
        o1_ref[rows, :] = out
        ya_ref[rows, :] = next_y(out + out0_ref[...])

    @pl.when(g == 1)
    def _():
        out = conv(ya_ref, _sigmoid)
        o2_ref[rows, :] = out
        yb_ref[rows, :] = next_y(out + o1_ref[rows, :])

    @pl.when(g == 2)
    def _():
        out = conv(yb_ref, _relu)
        ya_ref[rows, :] = next_y(out + out0_ref[...] + o2_ref[rows, :])

    @pl.when(g == 3)
    def _():
        o_ref[...] = conv(ya_ref, _elu)


def _mega(adj_bf16, y1, out0, ws_next, bs_tail):
    return pl.pallas_call(
        _mega_kernel,
        grid=(4, N // _BI),
        in_specs=[
            pl.BlockSpec(memory_space=pltpu.MemorySpace.HBM),
            pl.BlockSpec((N, D), lambda g, i: (0, 0)),
            pl.BlockSpec((_BI, D), lambda g, i: (i, 0)),
            pl.BlockSpec((1, D, D), lambda g, i: (jnp.minimum(g + 2, 4), 0, 0)),
            pl.BlockSpec((1, 1, D), lambda g, i: (g + 1, 0, 0)),
        ],
        out_specs=pl.BlockSpec(
            (_BI, D), lambda g, i: (jnp.where(g == 3, i, 0), 0)),
        out_shape=jax.ShapeDtypeStruct((N, D), jnp.float32),
        scratch_shapes=[
            pltpu.VMEM((3, _BI, N), jnp.bfloat16),
            pltpu.SemaphoreType.DMA((3,)),
            pltpu.VMEM((N, D), jnp.bfloat16),
            pltpu.VMEM((N, D), jnp.bfloat16),
            pltpu.VMEM((N, D), jnp.float32),
            pltpu.VMEM((N, D), jnp.float32),
        ],
        compiler_params=pltpu.CompilerParams(
            dimension_semantics=("arbitrary", "arbitrary")),
    )(adj_bf16, y1, out0, ws_next, bs_tail)


def kernel(x, adj, W1, b1, W2, b2, W3, b3, W4, b4, W5, b5):
    ws = jnp.stack([W1, W2, W3, W4, W5])
    bs = jnp.stack([b1, b2, b3, b4, b5]).reshape(5, 1, D)
    y0 = _y0(x, W1)
    out0, adj_bf16, y1 = _layer0(adj, y0, b1.reshape(1, D), W2)
    return _mega(adj_bf16, y1, out0, ws, bs)


# speedup vs baseline: 1.0182x; 1.0182x over previous
import functools

import jax
import jax.numpy as jnp
from jax.experimental import pallas as pl
from jax.experimental.pallas import tpu as pltpu

N = 10000
D = 128

_BI = 400
_BR = 2000


def _relu(h):
    return jnp.maximum(h, 0.0)


def _elu(h):
    return jnp.where(h > 0, h, jnp.exp(jnp.minimum(h, 0.0)) - 1.0)


def _sigmoid(h):
    return jax.nn.sigmoid(h)


def _layer0_kernel(adj_ref, x_ref, w1_ref, b_ref, w2_ref,
                   o_ref, adjb_ref, y1_ref, y0_ref):
    @pl.when(pl.program_id(0) == 0)
    def _():
        y0_ref[...] = jnp.dot(
            _relu(x_ref[...]), w1_ref[...], preferred_element_type=jnp.float32
        ).astype(jnp.bfloat16)

    ab = adj_ref[...].astype(jnp.bfloat16)
    adjb_ref[...] = ab
    h = jnp.dot(ab, y0_ref[...], preferred_element_type=jnp.float32)
    out = _relu(h + b_ref[...])
    o_ref[...] = out
    y1_ref[...] = jnp.dot(
        out, w2_ref[...], preferred_element_type=jnp.float32
    ).astype(jnp.bfloat16)


def _layer0(adj, x, w1, b1_2d, w2):
    return pl.pallas_call(
        _layer0_kernel,
        grid=(N // _BI,),
        in_specs=[
            pl.BlockSpec((_BI, N), lambda i: (i, 0)),
            pl.BlockSpec((N, D), lambda i: (0, 0)),
            pl.BlockSpec((D, D), lambda i: (0, 0)),
            pl.BlockSpec((1, D), lambda i: (0, 0)),
            pl.BlockSpec((D, D), lambda i: (0, 0)),
        ],
        out_specs=[
            pl.BlockSpec((_BI, D), lambda i: (i, 0)),
            pl.BlockSpec((_BI, N), lambda i: (i, 0)),
            pl.BlockSpec((_BI, D), lambda i: (i, 0)),
        ],
        out_shape=[
            jax.ShapeDtypeStruct((N, D), jnp.float32),
            jax.ShapeDtypeStruct((N, N), jnp.bfloat16),
            jax.ShapeDtypeStruct((N, D), jnp.bfloat16),
        ],
        scratch_shapes=[pltpu.VMEM((N, D), jnp.bfloat16)],
        compiler_params=pltpu.CompilerParams(
            dimension_semantics=("arbitrary",)),
    )(adj, x, w1, b1_2d, w2)


def _mega_kernel(adjb_ref, y1_ref, out0_ref, wn_ref, b_ref, o_ref,
                 bufs_ref, sems_ref, ya_ref, yb_ref, o1_ref, o2_ref):
    g = pl.program_id(0)
    i = pl.program_id(1)
    nb = N // _BI
    t = g * nb + i
    rows = pl.ds(i * _BI, _BI)

    def copy_for(tt):
        return pltpu.make_async_copy(
            adjb_ref.at[pl.ds((tt % nb) * _BI, _BI), :],
            bufs_ref.at[tt % 3],
            sems_ref.at[tt % 3],
        )

    @pl.when(t == 0)
    def _():
        copy_for(0).start()
        copy_for(1).start()

    @pl.when(t + 2 < 4 * nb)
    def _():
        copy_for(t + 2).start()

    copy_for(t).wait()
    slot = t % 3

    def conv(y_src, act):
        h = jnp.dot(bufs_ref[slot], y_src[...],
                    preferred_element_type=jnp.float32)
        return act(h + b_ref[0])

    def next_y(merge):
        return jnp.dot(_relu(merge), wn_ref[0],
                       preferred_element_type=jnp.float32).astype(jnp.bfloat16)

    @pl.when(g == 0)
    def _():
        out = conv(y1_ref, _elu)
        o1_ref[rows, :] = out
        ya_ref[rows, :] = next_y(out + out0_ref[...])

    @pl.when(g == 1)
    def _():
        out = conv(ya_ref, _sigmoid)
        o2_ref[rows, :] = out
        yb_ref[rows, :] = next_y(out + o1_ref[rows, :])

    @pl.when(g == 2)
    def _():
        out = conv(yb_ref, _relu)
        ya_ref[rows, :] = next_y(out + out0_ref[...] + o2_ref[rows, :])

    @pl.when(g == 3)
    def _():
        o_ref[...] = conv(ya_ref, _elu)


def _mega(adj_bf16, y1, out0, ws_next, bs_tail):
    return pl.pallas_call(
        _mega_kernel,
        grid=(4, N // _BI),
        in_specs=[
            pl.BlockSpec(memory_space=pltpu.MemorySpace.HBM),
            pl.BlockSpec((N, D), lambda g, i: (0, 0)),
            pl.BlockSpec((_BI, D), lambda g, i: (i, 0)),
            pl.BlockSpec((1, D, D), lambda g, i: (jnp.minimum(g + 2, 4), 0, 0)),
            pl.BlockSpec((1, 1, D), lambda g, i: (g + 1, 0, 0)),
        ],
        out_specs=pl.BlockSpec(
            (_BI, D), lambda g, i: (jnp.where(g == 3, i, 0), 0)),
        out_shape=jax.ShapeDtypeStruct((N, D), jnp.float32),
        scratch_shapes=[
            pltpu.VMEM((3, _BI, N), jnp.bfloat16),
            pltpu.SemaphoreType.DMA((3,)),
            pltpu.VMEM((N, D), jnp.bfloat16),
            pltpu.VMEM((N, D), jnp.bfloat16),
            pltpu.VMEM((N, D), jnp.float32),
            pltpu.VMEM((N, D), jnp.float32),
        ],
        compiler_params=pltpu.CompilerParams(
            dimension_semantics=("arbitrary", "arbitrary")),
    )(adj_bf16, y1, out0, ws_next, bs_tail)


def kernel(x, adj, W1, b1, W2, b2, W3, b3, W4, b4, W5, b5):
    ws = jnp.stack([W1, W2, W3, W4, W5])
    bs = jnp.stack([b1, b2, b3, b4, b5]).reshape(5, 1, D)
    out0, adj_bf16, y1 = _layer0(adj, x, W1, b1.reshape(1, D), W2)
    return _mega(adj_bf16, y1, out0, ws, bs)
